# baseline (speedup 1.0000x reference)
import functools

import jax
import jax.numpy as jnp
from jax import lax
from jax.experimental import pallas as pl
from jax.experimental.pallas import tpu as pltpu
from jax.experimental.pallas import tpu_sc as plsc

_RATIO = 3
_MIN_NEG = 0
_R = 64
_C = 20000
_NW = 32
_RPW = _R // _NW
_NCH = _C // 16
_INT_MIN = -2147483648

_MESH = plsc.VectorSubcoreMesh(core_axis_name="c", subcore_axis_name="s")
_CP = pltpu.CompilerParams(needs_layout_passes=False)

_NB1 = 2048
_NB2 = 2048
_NB3 = 1024
_H2W = _NB2 + 16
_H3W = _NB3 + 16


def _wid():
    return lax.axis_index("s") * 2 + lax.axis_index("c")


def _tree_sum(vals):
    while len(vals) > 1:
        vals = [a + b for a, b in zip(vals[::2], vals[1::2])]
    return vals[0]


def _merge_hists(hall_v, mrg_v, nbins):
    def merge(j, _):
        sl = pl.ds(j * 16, 16)
        mrg_v[sl] = _tree_sum([hall_v[w2, sl] for w2 in range(_NW)])
        return 0
    lax.fori_loop(0, nbins // 16, merge, 0)


_SUB = 5


def _zero_subhists(h_v, width):
    def zh(i, _):
        for u in range(_SUB):
            h_v[pl.ds(u * width + i * 16, 16)] = jnp.zeros((16,), jnp.float32)
        return 0
    lax.fori_loop(0, width // 16, zh, 0)


def _fold_subhists(h_v, mh_v, width):
    def fold(i, _):
        sl = pl.ds(i * 16, 16)
        mh_v[sl] = _tree_sum(
            [h_v[pl.ds(u * width + i * 16, 16)] for u in range(_SUB)])
        return 0
    lax.fori_loop(0, width // 16, fold, 0)


def _keys_from(v, nmv):
    flat = v * nmv.astype(jnp.float32)
    b = plsc.bitcast(flat, jnp.int32)
    key = jnp.where(b >= 0, b, b ^ jnp.int32(0x7FFFFFFF))
    return flat, key


def _scan_bins(merged_ref, nbins, k_rem):
    nch = nbins // 16
    kf = k_rem.astype(jnp.float32)

    def body(t, carry):
        acc, found, bsel, above = carry
        base = nbins - 16 * (t + 1)
        h = merged_ref[pl.ds(base, 16)]
        cs = plsc.cumsum(h)
        chunk_sum = cs[15]
        suff = acc + (chunk_sum - cs) + h
        m = suff >= kf
        pc = plsc.all_reduce_population_count(m)[0]
        has = pc > 0
        lane = pc - 1
        csl = jnp.max(jnp.where(m, cs, 0.0))
        hit = jnp.logical_and(has, jnp.logical_not(found))
        bsel = jnp.where(hit, base + lane, bsel)
        above = jnp.where(hit, acc + chunk_sum - csl, above)
        found = jnp.logical_or(found, has)
        return acc + chunk_sum, found, bsel, above

    _, _, bsel, above = lax.fori_loop(
        0, nch, body,
        (jnp.float32(0.0), jnp.bool_(False), jnp.int32(0), jnp.float32(0.0)))
    return bsel, above


def _lane_pack(pairs):
    io = lax.iota(jnp.int32, 16)
    out = jnp.zeros((16,), jnp.int32)
    for lane, val in pairs:
        out = jnp.where(io == lane, val, out)
    return out


@functools.partial(
    pl.kernel,
    out_type=(jax.ShapeDtypeStruct((_R, _C), jnp.int32),
              jax.ShapeDtypeStruct((_NW, 16), jnp.float32),
              jax.ShapeDtypeStruct((_NW, 16), jnp.int32),
              jax.ShapeDtypeStruct((_NW, _NB1), jnp.float32)),
    mesh=_MESH,
    scratch_types=[pltpu.VMEM((_C,), jnp.float32),
                   pltpu.VMEM((_C,), jnp.int32),
                   pltpu.VMEM((_C,), jnp.int32),
                   pltpu.VMEM((_SUB * _NB1,), jnp.float32),
                   pltpu.VMEM((_NB1,), jnp.float32),
                   pltpu.VMEM((16,), jnp.float32),
                   pltpu.VMEM((16,), jnp.int32)],
    compiler_params=_CP,
)
def _c1(v_hbm, pm_hbm, nm_hbm, keys_hbm, posrow_hbm, counts_hbm, hist1_hbm,
        v_v, pm_v, nm_v, h_v, mh_v, pr_v, ct_v):
    w = _wid()
    _zero_subhists(h_v, _NB1)

    ones = jnp.ones((16,), jnp.float32)
    psums = []
    npos_t = jnp.int32(0)
    nnz_t = jnp.int32(0)
    for r in range(_RPW):
        row = w * _RPW + r
        pltpu.sync_copy(v_hbm.at[row], v_v)
        pltpu.sync_copy(pm_hbm.at[row], pm_v)
        pltpu.sync_copy(nm_hbm.at[row], nm_v)

        def body(i, carry):
            psum, npos, nnz = carry
            for u in range(_SUB):
                sl = pl.ds(i * (16 * _SUB) + u * 16, 16)
                v = v_v[sl]
                pmv = pm_v[sl]
                nmv = nm_v[sl]
                psum = psum + v * pmv.astype(jnp.float32)
                flat, key = _keys_from(v, nmv)
                nm_v[sl] = key
                nnz = nnz + (flat != 0.0).astype(jnp.int32)
                bin1 = jnp.right_shift(key, 21) + 1024 + (u * _NB1)
                plsc.addupdate_scatter(h_v, [bin1], ones)
                npos = npos + pmv
            return psum, npos, nnz

        psum, nposv, nnzv = lax.fori_loop(
            0, _NCH // _SUB, body,
            (jnp.zeros((16,), jnp.float32), jnp.zeros((16,), jnp.int32),
             jnp.zeros((16,), jnp.int32)))
        pltpu.sync_copy(nm_v, keys_hbm.at[row])
        psums.append((r, jnp.sum(psum)))
        npos_t = npos_t + jnp.sum(nposv)
        nnz_t = nnz_t + jnp.sum(nnzv)
    _fold_subhists(h_v, mh_v, _NB1)

    io = lax.iota(jnp.int32, 16)
    prv = jnp.zeros((16,), jnp.float32)
    for r, s in psums:
        prv = jnp.where(io == r, s, prv)
    pr_v[...] = prv
    ct_v[...] = _lane_pack([(0, npos_t), (1, nnz_t)])
    pltpu.sync_copy(pr_v, posrow_hbm.at[w])
    pltpu.sync_copy(ct_v, counts_hbm.at[w])
    pltpu.sync_copy(mh_v, hist1_hbm.at[w])


@functools.partial(
    pl.kernel,
    out_type=(jax.ShapeDtypeStruct((_NW, _H2W), jnp.float32),
              jax.ShapeDtypeStruct((_NW, 16), jnp.int32)),
    mesh=_MESH,
    scratch_types=[pltpu.VMEM((_NW, _NB1), jnp.float32),
                   pltpu.VMEM((_NB1,), jnp.float32),
                   pltpu.VMEM((_NW, 16), jnp.int32),
                   pltpu.VMEM((_C,), jnp.int32),
                   pltpu.VMEM((_SUB * _H2W,), jnp.float32),
                   pltpu.VMEM((_H2W,), jnp.float32),
                   pltpu.VMEM((16,), jnp.int32)],
    compiler_params=_CP,
)
def _c2(counts_hbm, hist1_hbm, keys_hbm, hist2_hbm, sel2_hbm,
        hall_v, mrg_v, ct_v, k_v, h_v, mh_v, sel_v):
    w = _wid()
    pltpu.sync_copy(hist1_hbm, hall_v)
    pltpu.sync_copy(counts_hbm, ct_v)

    _merge_hists(hall_v, mrg_v, _NB1)

    cts = _tree_sum([ct_v[w2, :] for w2 in range(_NW)])
    npos = cts[0]
    nnz = cts[1]
    k_tot = jnp.minimum(
        jnp.maximum(jnp.int32(_RATIO) * npos, jnp.int32(_MIN_NEG)), nnz)

    b1, above1 = _scan_bins(mrg_v, _NB1, k_tot)
    k_rem = k_tot - above1.astype(jnp.int32)

    _zero_subhists(h_v, _H2W)

    ones = jnp.ones((16,), jnp.float32)
    for r in range(_RPW):
        row = w * _RPW + r
        pltpu.sync_copy(keys_hbm.at[row], k_v)

        def body(i, _):
            for u in range(_SUB):
                key = k_v[pl.ds(i * (16 * _SUB) + u * 16, 16)]
                match = (jnp.right_shift(key, 21) + 1024) == b1
                bin2 = jnp.right_shift(key, 10) & jnp.int32(0x7FF)
                idx = jnp.where(match, bin2, jnp.int32(_NB2)) + (u * _H2W)
                plsc.addupdate_scatter(h_v, [idx], ones)
            return 0
        lax.fori_loop(0, _NCH // _SUB, body, 0)
    _fold_subhists(h_v, mh_v, _H2W)

    sel_v[...] = _lane_pack([(0, b1), (1, k_rem), (3, k_tot)])
    pltpu.sync_copy(mh_v, hist2_hbm.at[w])
    pltpu.sync_copy(sel_v, sel2_hbm.at[w])


@functools.partial(
    pl.kernel,
    out_type=(jax.ShapeDtypeStruct((_NW, _H3W), jnp.float32),
              jax.ShapeDtypeStruct((_NW, 16), jnp.int32)),
    mesh=_MESH,
    scratch_types=[pltpu.VMEM((_NW, _H2W), jnp.float32),
                   pltpu.VMEM((_NB2,), jnp.float32),
                   pltpu.VMEM((16,), jnp.int32),
                   pltpu.VMEM((_C,), jnp.int32),
                   pltpu.VMEM((_SUB * _H3W,), jnp.float32),
                   pltpu.VMEM((_H3W,), jnp.float32),
                   pltpu.VMEM((16,), jnp.int32)],
    compiler_params=_CP,
)
def _c3(sel2_hbm, hist2_hbm, keys_hbm, hist3_hbm, sel3_hbm,
        hall_v, mrg_v, s_v, k_v, h_v, mh_v, sel_v):
    w = _wid()
    pltpu.sync_copy(hist2_hbm, hall_v)
    pltpu.sync_copy(sel2_hbm.at[0], s_v)
    sel = s_v[...]
    b1 = sel[0]
    k_in = sel[1]
    k_tot = sel[3]

    _merge_hists(hall_v, mrg_v, _NB2)

    b2, above2 = _scan_bins(mrg_v, _NB2, k_in)
    k_rem = k_in - above2.astype(jnp.int32)
    top22s = jnp.left_shift(b1 - 1024, 11) + b2

    _zero_subhists(h_v, _H3W)

    ones = jnp.ones((16,), jnp.float32)
    for r in range(_RPW):
        row = w * _RPW + r
        pltpu.sync_copy(keys_hbm.at[row], k_v)

        def body(i, _):
            for u in range(_SUB):
                key = k_v[pl.ds(i * (16 * _SUB) + u * 16, 16)]
                match = jnp.right_shift(key, 10) == top22s
                bin3 = key & jnp.int32(0x3FF)
                idx = jnp.where(match, bin3, jnp.int32(_NB3)) + (u * _H3W)
                plsc.addupdate_scatter(h_v, [idx], ones)
            return 0
        lax.fori_loop(0, _NCH // _SUB, body, 0)
    _fold_subhists(h_v, mh_v, _H3W)

    sel_v[...] = _lane_pack([(0, top22s), (1, k_rem), (3, k_tot)])
    pltpu.sync_copy(mh_v, hist3_hbm.at[w])
    pltpu.sync_copy(sel_v, sel3_hbm.at[w])


@functools.partial(
    pl.kernel,
    out_type=(jax.ShapeDtypeStruct((_NW, 16), jnp.int32),
              jax.ShapeDtypeStruct((_NW, 16), jnp.int32)),
    mesh=_MESH,
    scratch_types=[pltpu.VMEM((_NW, _H3W), jnp.float32),
                   pltpu.VMEM((_NB3,), jnp.float32),
                   pltpu.VMEM((16,), jnp.int32),
                   pltpu.VMEM((_C,), jnp.int32),
                   pltpu.VMEM((16,), jnp.int32),
                   pltpu.VMEM((16,), jnp.int32)],
    compiler_params=_CP,
)
def _c4(sel3_hbm, hist3_hbm, keys_hbm, rowties_hbm, sel4_hbm,
        hall_v, mrg_v, s_v, k_v, rt_v, sel_v):
    w = _wid()
    pltpu.sync_copy(hist3_hbm, hall_v)
    pltpu.sync_copy(sel3_hbm.at[0], s_v)
    sel = s_v[...]
    top22s = sel[0]
    k_in = sel[1]
    k_tot = sel[3]

    _merge_hists(hall_v, mrg_v, _NB3)

    b3, above3 = _scan_bins(mrg_v, _NB3, k_in)
    c_take = k_in - above3.astype(jnp.int32)
    t_key = jnp.left_shift(top22s, 10) | b3

    ties = []
    for r in range(_RPW):
        row = w * _RPW + r
        pltpu.sync_copy(keys_hbm.at[row], k_v)

        def body(i, acc):
            for u in range(2):
                key = k_v[pl.ds(i * 32 + u * 16, 16)]
                acc = acc + (key == t_key).astype(jnp.int32)
            return acc
        tv = lax.fori_loop(0, _NCH // 2, body, jnp.zeros((16,), jnp.int32))
        ties.append((r, jnp.sum(tv)))

    rt_v[...] = _lane_pack(ties)
    sel_v[...] = _lane_pack([(0, t_key), (1, c_take), (3, k_tot)])
    pltpu.sync_copy(rt_v, rowties_hbm.at[w])
    pltpu.sync_copy(sel_v, sel4_hbm.at[w])


@functools.partial(
    pl.kernel,
    out_type=jax.ShapeDtypeStruct((_NW, 16), jnp.float32),
    mesh=_MESH,
    scratch_types=[pltpu.VMEM((_NW, 16), jnp.int32),
                   pltpu.VMEM((16,), jnp.int32),
                   pltpu.VMEM((16,), jnp.float32),
                   pltpu.VMEM((_C,), jnp.int32),
                   pltpu.VMEM((_C,), jnp.float32),
                   pltpu.VMEM((16,), jnp.float32)],
    compiler_params=_CP,
)
def _c5(sel4_hbm, rowties_hbm, posrow_hbm, keys_hbm, v_hbm, out_hbm,
        rt_v, s_v, pr_v, k_v, v_v, o_v):
    w = _wid()
    pltpu.sync_copy(rowties_hbm, rt_v)
    pltpu.sync_copy(sel4_hbm.at[0], s_v)
    pltpu.sync_copy(posrow_hbm.at[w], pr_v)
    sel = s_v[...]
    t_key = sel[0]
    c_take = sel[1]

    def pre(w2, acc):
        return acc + rt_v[w2, :]
    prev = lax.fori_loop(0, w, pre, jnp.zeros((16,), jnp.int32))
    excl0 = prev[0] + prev[1]
    own = rt_v[w, :]
    rt0 = own[0]
    rt1 = own[1]
    take0 = jnp.clip(c_take - excl0, 0, rt0)
    take1 = jnp.clip(c_take - (excl0 + rt0), 0, rt1)
    takes = (take0, take1)
    pr = pr_v[...]

    outs = []
    for r in range(_RPW):
        row = w * _RPW + r
        pltpu.sync_copy(keys_hbm.at[row], k_v)
        pltpu.sync_copy(v_hbm.at[row], v_v)
        take_r = takes[r]

        def body(i, carry):
            negacc, run = carry
            for u in range(2):
                sl = pl.ds(i * 32 + u * 16, 16)
                key = k_v[sl]
                v = v_v[sl]
                tie = key == t_key
                tiei = tie.astype(jnp.int32)
                incl = plsc.cumsum(tiei)
                rank = run + incl - tiei
                keep = (key > t_key) | (tie & (rank < take_r))
                negacc = negacc + jnp.where(keep, v, 0.0)
                run = run + incl[15]
            return negacc, run

        negacc, _ = lax.fori_loop(
            0, _NCH // 2, body, (jnp.zeros((16,), jnp.float32), jnp.int32(0)))
        outs.append((r, pr[r] + jnp.sum(negacc)))

    io = lax.iota(jnp.int32, 16)
    ov = jnp.zeros((16,), jnp.float32)
    for r, s in outs:
        ov = jnp.where(io == r, s, ov)
    o_v[...] = ov
    pltpu.sync_copy(o_v, out_hbm.at[w])


def kernel(values, positive_mask, negative_mask):
    keys, posrow, counts, hist1 = _c1(values, positive_mask, negative_mask)
    hist2, sel2 = _c2(counts, hist1, keys)
    hist3, sel3 = _c3(sel2, hist2, keys)
    rowties, sel4 = _c4(sel3, hist3, keys)
    out = _c5(sel4, rowties, posrow, keys, values)
    return out[:, :_RPW].reshape(_R)


# speedup vs baseline: 1.0051x; 1.0051x over previous
import functools

import jax
import jax.numpy as jnp
from jax import lax
from jax.experimental import pallas as pl
from jax.experimental.pallas import tpu as pltpu
from jax.experimental.pallas import tpu_sc as plsc

_RATIO = 3
_MIN_NEG = 0
_R = 64
_C = 20000
_NW = 32
_RPW = _R // _NW
_NCH = _C // 16
_INT_MIN = -2147483648

_MESH = plsc.VectorSubcoreMesh(core_axis_name="c", subcore_axis_name="s")
_CP = pltpu.CompilerParams(needs_layout_passes=False)

_NB1 = 2048
_NB2 = 2048
_NB3 = 1024
_H2W = _NB2 + 16
_H3W = _NB3 + 16


def _wid():
    return lax.axis_index("s") * 2 + lax.axis_index("c")


def _tree_sum(vals):
    while len(vals) > 1:
        vals = [a + b for a, b in zip(vals[::2], vals[1::2])]
    return vals[0]


def _merge_hists(hall_v, mrg_v, nbins):
    def merge(j, _):
        sl = pl.ds(j * 16, 16)
        mrg_v[sl] = _tree_sum([hall_v[w2, sl] for w2 in range(_NW)])
        return 0
    lax.fori_loop(0, nbins // 16, merge, 0)


def _zero_hist(h_v, width):
    def zh(i, _):
        h_v[pl.ds(i * 16, 16)] = jnp.zeros((16,), jnp.float32)
        return 0
    lax.fori_loop(0, width // 16, zh, 0)


def _keys_from(v, nmv):
    flat = v * nmv.astype(jnp.float32)
    b = plsc.bitcast(flat, jnp.int32)
    key = jnp.where(b >= 0, b, b ^ jnp.int32(0x7FFFFFFF))
    return flat, key


def _scan_bins(merged_ref, nbins, k_rem):
    nch = nbins // 16
    kf = k_rem.astype(jnp.float32)

    def body(t, carry):
        acc, found, bsel, above = carry
        base = nbins - 16 * (t + 1)
        h = merged_ref[pl.ds(base, 16)]
        cs = plsc.cumsum(h)
        chunk_sum = cs[15]
        suff = acc + (chunk_sum - cs) + h
        m = suff >= kf
        pc = plsc.all_reduce_population_count(m)[0]
        has = pc > 0
        lane = pc - 1
        csl = jnp.max(jnp.where(m, cs, 0.0))
        hit = jnp.logical_and(has, jnp.logical_not(found))
        bsel = jnp.where(hit, base + lane, bsel)
        above = jnp.where(hit, acc + chunk_sum - csl, above)
        found = jnp.logical_or(found, has)
        return acc + chunk_sum, found, bsel, above

    _, _, bsel, above = lax.fori_loop(
        0, nch, body,
        (jnp.float32(0.0), jnp.bool_(False), jnp.int32(0), jnp.float32(0.0)))
    return bsel, above


def _lane_pack(pairs):
    io = lax.iota(jnp.int32, 16)
    out = jnp.zeros((16,), jnp.int32)
    for lane, val in pairs:
        out = jnp.where(io == lane, val, out)
    return out


@functools.partial(
    pl.kernel,
    out_type=(jax.ShapeDtypeStruct((_R, _C), jnp.int32),
              jax.ShapeDtypeStruct((_NW, 16), jnp.float32),
              jax.ShapeDtypeStruct((_NW, 16), jnp.int32),
              jax.ShapeDtypeStruct((_NW, _NB1), jnp.float32)),
    mesh=_MESH,
    scratch_types=[pltpu.VMEM((_C,), jnp.float32),
                   pltpu.VMEM((_C,), jnp.int32),
                   pltpu.VMEM((_C,), jnp.int32),
                   pltpu.VMEM((_NB1,), jnp.float32),
                   pltpu.VMEM((16,), jnp.float32),
                   pltpu.VMEM((16,), jnp.int32)],
    compiler_params=_CP,
)
def _c1(v_hbm, pm_hbm, nm_hbm, keys_hbm, posrow_hbm, counts_hbm, hist1_hbm,
        v_v, pm_v, nm_v, h_v, pr_v, ct_v):
    w = _wid()
    _zero_hist(h_v, _NB1)

    ones = jnp.ones((16,), jnp.float32)
    psums = []
    npos_t = jnp.int32(0)
    nnz_t = jnp.int32(0)
    for r in range(_RPW):
        row = w * _RPW + r
        pltpu.sync_copy(v_hbm.at[row], v_v)
        pltpu.sync_copy(pm_hbm.at[row], pm_v)
        pltpu.sync_copy(nm_hbm.at[row], nm_v)

        def body(i, carry):
            psum, npos, nnz = carry
            for u in range(5):
                sl = pl.ds(i * 80 + u * 16, 16)
                v = v_v[sl]
                pmv = pm_v[sl]
                nmv = nm_v[sl]
                psum = psum + v * pmv.astype(jnp.float32)
                flat, key = _keys_from(v, nmv)
                nm_v[sl] = key
                nnz = nnz + (flat != 0.0).astype(jnp.int32)
                bin1 = jnp.right_shift(key, 21) + 1024
                plsc.addupdate_scatter(h_v, [bin1], ones)
                npos = npos + pmv
            return psum, npos, nnz

        psum, nposv, nnzv = lax.fori_loop(
            0, _NCH // 5, body,
            (jnp.zeros((16,), jnp.float32), jnp.zeros((16,), jnp.int32),
             jnp.zeros((16,), jnp.int32)))
        pltpu.sync_copy(nm_v, keys_hbm.at[row])
        psums.append((r, jnp.sum(psum)))
        npos_t = npos_t + jnp.sum(nposv)
        nnz_t = nnz_t + jnp.sum(nnzv)

    io = lax.iota(jnp.int32, 16)
    prv = jnp.zeros((16,), jnp.float32)
    for r, s in psums:
        prv = jnp.where(io == r, s, prv)
    pr_v[...] = prv
    ct_v[...] = _lane_pack([(0, npos_t), (1, nnz_t)])
    pltpu.sync_copy(pr_v, posrow_hbm.at[w])
    pltpu.sync_copy(ct_v, counts_hbm.at[w])
    pltpu.sync_copy(h_v, hist1_hbm.at[w])


@functools.partial(
    pl.kernel,
    out_type=(jax.ShapeDtypeStruct((_NW, _H2W), jnp.float32),
              jax.ShapeDtypeStruct((_NW, 16), jnp.int32)),
    mesh=_MESH,
    scratch_types=[pltpu.VMEM((_NW, _NB1), jnp.float32),
                   pltpu.VMEM((_NB1,), jnp.float32),
                   pltpu.VMEM((_NW, 16), jnp.int32),
                   pltpu.VMEM((_C,), jnp.int32),
                   pltpu.VMEM((_H2W,), jnp.float32),
                   pltpu.VMEM((16,), jnp.int32)],
    compiler_params=_CP,
)
def _c2(counts_hbm, hist1_hbm, keys_hbm, hist2_hbm, sel2_hbm,
        hall_v, mrg_v, ct_v, k_v, h_v, sel_v):
    w = _wid()
    pltpu.sync_copy(hist1_hbm, hall_v)
    pltpu.sync_copy(counts_hbm, ct_v)

    _merge_hists(hall_v, mrg_v, _NB1)

    cts = _tree_sum([ct_v[w2, :] for w2 in range(_NW)])
    npos = cts[0]
    nnz = cts[1]
    k_tot = jnp.minimum(
        jnp.maximum(jnp.int32(_RATIO) * npos, jnp.int32(_MIN_NEG)), nnz)

    b1, above1 = _scan_bins(mrg_v, _NB1, k_tot)
    k_rem = k_tot - above1.astype(jnp.int32)

    _zero_hist(h_v, _H2W)

    ones = jnp.ones((16,), jnp.float32)
    for r in range(_RPW):
        row = w * _RPW + r
        pltpu.sync_copy(keys_hbm.at[row], k_v)

        def body(i, _):
            for u in range(5):
                key = k_v[pl.ds(i * 80 + u * 16, 16)]
                match = (jnp.right_shift(key, 21) + 1024) == b1
                bin2 = jnp.right_shift(key, 10) & jnp.int32(0x7FF)
                idx = jnp.where(match, bin2, jnp.int32(_NB2))
                plsc.addupdate_scatter(h_v, [idx], ones)
            return 0
        lax.fori_loop(0, _NCH // 5, body, 0)

    sel_v[...] = _lane_pack([(0, b1), (1, k_rem), (3, k_tot)])
    pltpu.sync_copy(h_v, hist2_hbm.at[w])
    pltpu.sync_copy(sel_v, sel2_hbm.at[w])


@functools.partial(
    pl.kernel,
    out_type=(jax.ShapeDtypeStruct((_NW, _H3W), jnp.float32),
              jax.ShapeDtypeStruct((_NW, 16), jnp.int32)),
    mesh=_MESH,
    scratch_types=[pltpu.VMEM((_NW, _H2W), jnp.float32),
                   pltpu.VMEM((_NB2,), jnp.float32),
                   pltpu.VMEM((16,), jnp.int32),
                   pltpu.VMEM((_C,), jnp.int32),
                   pltpu.VMEM((_H3W,), jnp.float32),
                   pltpu.VMEM((16,), jnp.int32)],
    compiler_params=_CP,
)
def _c3(sel2_hbm, hist2_hbm, keys_hbm, hist3_hbm, sel3_hbm,
        hall_v, mrg_v, s_v, k_v, h_v, sel_v):
    w = _wid()
    pltpu.sync_copy(hist2_hbm, hall_v)
    pltpu.sync_copy(sel2_hbm.at[0], s_v)
    sel = s_v[...]
    b1 = sel[0]
    k_in = sel[1]
    k_tot = sel[3]

    _merge_hists(hall_v, mrg_v, _NB2)

    b2, above2 = _scan_bins(mrg_v, _NB2, k_in)
    k_rem = k_in - above2.astype(jnp.int32)
    top22s = jnp.left_shift(b1 - 1024, 11) + b2

    _zero_hist(h_v, _H3W)

    ones = jnp.ones((16,), jnp.float32)
    for r in range(_RPW):
        row = w * _RPW + r
        pltpu.sync_copy(keys_hbm.at[row], k_v)

        def body(i, _):
            for u in range(5):
                key = k_v[pl.ds(i * 80 + u * 16, 16)]
                match = jnp.right_shift(key, 10) == top22s
                bin3 = key & jnp.int32(0x3FF)
                idx = jnp.where(match, bin3, jnp.int32(_NB3))
                plsc.addupdate_scatter(h_v, [idx], ones)
            return 0
        lax.fori_loop(0, _NCH // 5, body, 0)

    sel_v[...] = _lane_pack([(0, top22s), (1, k_rem), (3, k_tot)])
    pltpu.sync_copy(h_v, hist3_hbm.at[w])
    pltpu.sync_copy(sel_v, sel3_hbm.at[w])


@functools.partial(
    pl.kernel,
    out_type=(jax.ShapeDtypeStruct((_NW, 16), jnp.int32),
              jax.ShapeDtypeStruct((_NW, 16), jnp.int32)),
    mesh=_MESH,
    scratch_types=[pltpu.VMEM((_NW, _H3W), jnp.float32),
                   pltpu.VMEM((_NB3,), jnp.float32),
                   pltpu.VMEM((16,), jnp.int32),
                   pltpu.VMEM((_C,), jnp.int32),
                   pltpu.VMEM((16,), jnp.int32),
                   pltpu.VMEM((16,), jnp.int32)],
    compiler_params=_CP,
)
def _c4(sel3_hbm, hist3_hbm, keys_hbm, rowties_hbm, sel4_hbm,
        hall_v, mrg_v, s_v, k_v, rt_v, sel_v):
    w = _wid()
    pltpu.sync_copy(hist3_hbm, hall_v)
    pltpu.sync_copy(sel3_hbm.at[0], s_v)
    sel = s_v[...]
    top22s = sel[0]
    k_in = sel[1]
    k_tot = sel[3]

    _merge_hists(hall_v, mrg_v, _NB3)

    b3, above3 = _scan_bins(mrg_v, _NB3, k_in)
    c_take = k_in - above3.astype(jnp.int32)
    t_key = jnp.left_shift(top22s, 10) | b3

    ties = []
    for r in range(_RPW):
        row = w * _RPW + r
        pltpu.sync_copy(keys_hbm.at[row], k_v)

        def body(i, acc):
            for u in range(5):
                key = k_v[pl.ds(i * 80 + u * 16, 16)]
                acc = acc + (key == t_key).astype(jnp.int32)
            return acc
        tv = lax.fori_loop(0, _NCH // 5, body, jnp.zeros((16,), jnp.int32))
        ties.append((r, jnp.sum(tv)))

    rt_v[...] = _lane_pack(ties)
    sel_v[...] = _lane_pack([(0, t_key), (1, c_take), (3, k_tot)])
    pltpu.sync_copy(rt_v, rowties_hbm.at[w])
    pltpu.sync_copy(sel_v, sel4_hbm.at[w])


@functools.partial(
    pl.kernel,
    out_type=jax.ShapeDtypeStruct((_NW, 16), jnp.float32),
    mesh=_MESH,
    scratch_types=[pltpu.VMEM((_NW, 16), jnp.int32),
                   pltpu.VMEM((16,), jnp.int32),
                   pltpu.VMEM((16,), jnp.float32),
                   pltpu.VMEM((_C,), jnp.int32),
                   pltpu.VMEM((_C,), jnp.float32),
                   pltpu.VMEM((16,), jnp.float32)],
    compiler_params=_CP,
)
def _c5(sel4_hbm, rowties_hbm, posrow_hbm, keys_hbm, v_hbm, out_hbm,
        rt_v, s_v, pr_v, k_v, v_v, o_v):
    w = _wid()
    pltpu.sync_copy(rowties_hbm, rt_v)
    pltpu.sync_copy(sel4_hbm.at[0], s_v)
    pltpu.sync_copy(posrow_hbm.at[w], pr_v)
    sel = s_v[...]
    t_key = sel[0]
    c_take = sel[1]

    def pre(w2, acc):
        return acc + rt_v[w2, :]
    prev = lax.fori_loop(0, w, pre, jnp.zeros((16,), jnp.int32))
    excl0 = prev[0] + prev[1]
    own = rt_v[w, :]
    rt0 = own[0]
    rt1 = own[1]
    take0 = jnp.clip(c_take - excl0, 0, rt0)
    take1 = jnp.clip(c_take - (excl0 + rt0), 0, rt1)
    takes = (take0, take1)
    pr = pr_v[...]

    outs = []
    for r in range(_RPW):
        row = w * _RPW + r
        pltpu.sync_copy(keys_hbm.at[row], k_v)
        pltpu.sync_copy(v_hbm.at[row], v_v)
        take_r = takes[r]

        def body(i, carry):
            negacc, run = carry
            for u in range(5):
                sl = pl.ds(i * 80 + u * 16, 16)
                key = k_v[sl]
                v = v_v[sl]
                tie = key == t_key
                tiei = tie.astype(jnp.int32)
                incl = plsc.cumsum(tiei)
                rank = run + incl - tiei
                keep = (key > t_key) | (tie & (rank < take_r))
                negacc = negacc + jnp.where(keep, v, 0.0)
                run = run + incl[15]
            return negacc, run

        negacc, _ = lax.fori_loop(
            0, _NCH // 5, body, (jnp.zeros((16,), jnp.float32), jnp.int32(0)))
        outs.append((r, pr[r] + jnp.sum(negacc)))

    io = lax.iota(jnp.int32, 16)
    ov = jnp.zeros((16,), jnp.float32)
    for r, s in outs:
        ov = jnp.where(io == r, s, ov)
    o_v[...] = ov
    pltpu.sync_copy(o_v, out_hbm.at[w])


def kernel(values, positive_mask, negative_mask):
    keys, posrow, counts, hist1 = _c1(values, positive_mask, negative_mask)
    hist2, sel2 = _c2(counts, hist1, keys)
    hist3, sel3 = _c3(sel2, hist2, keys)
    rowties, sel4 = _c4(sel3, hist3, keys)
    out = _c5(sel4, rowties, posrow, keys, values)
    return out[:, :_RPW].reshape(_R)


# speedup vs baseline: 1.2139x; 1.2077x over previous
import functools

import jax
import jax.numpy as jnp
from jax import lax
from jax.experimental import pallas as pl
from jax.experimental.pallas import tpu as pltpu
from jax.experimental.pallas import tpu_sc as plsc

_RATIO = 3
_MIN_NEG = 0
_R = 64
_C = 20000
_NW = 32
_RPW = _R // _NW
_NCH = _C // 16
_INT_MIN = -2147483648

_MESH = plsc.VectorSubcoreMesh(core_axis_name="c", subcore_axis_name="s")
_CP = pltpu.CompilerParams(needs_layout_passes=False)

_NB1 = 2048
_NB2 = 2048
_NB3 = 1024
_H2W = _NB2 + 16
_H3W = _NB3 + 16


def _wid():
    return lax.axis_index("s") * 2 + lax.axis_index("c")


def _tree_sum(vals):
    while len(vals) > 1:
        vals = [a + b for a, b in zip(vals[::2], vals[1::2])]
    return vals[0]


def _merge_hists(hall_v, mrg_v, nbins):
    def merge(j, _):
        sl = pl.ds(j * 16, 16)
        mrg_v[sl] = _tree_sum([hall_v[w2, sl] for w2 in range(_NW)])
        return 0
    lax.fori_loop(0, nbins // 16, merge, 0)


def _zero_hist(h_v, width):
    def zh(i, _):
        h_v[pl.ds(i * 16, 16)] = jnp.zeros((16,), jnp.float32)
        return 0
    lax.fori_loop(0, width // 16, zh, 0)


def _keys_from(v, nmv):
    flat = v * nmv.astype(jnp.float32)
    b = plsc.bitcast(flat, jnp.int32)
    key = jnp.where(b >= 0, b, b ^ jnp.int32(0x7FFFFFFF))
    return flat, key


def _scan_bins(merged_ref, nbins, k_rem):
    nch = nbins // 16
    kf = k_rem.astype(jnp.float32)

    def body(t, carry):
        acc, found, bsel, above = carry
        base = nbins - 16 * (t + 1)
        h = merged_ref[pl.ds(base, 16)]
        cs = plsc.cumsum(h)
        chunk_sum = cs[15]
        suff = acc + (chunk_sum - cs) + h
        m = suff >= kf
        pc = plsc.all_reduce_population_count(m)[0]
        has = pc > 0
        lane = pc - 1
        csl = jnp.max(jnp.where(m, cs, 0.0))
        hit = jnp.logical_and(has, jnp.logical_not(found))
        bsel = jnp.where(hit, base + lane, bsel)
        above = jnp.where(hit, acc + chunk_sum - csl, above)
        found = jnp.logical_or(found, has)
        return acc + chunk_sum, found, bsel, above

    _, _, bsel, above = lax.fori_loop(
        0, nch, body,
        (jnp.float32(0.0), jnp.bool_(False), jnp.int32(0), jnp.float32(0.0)))
    return bsel, above


def _lane_pack(pairs):
    io = lax.iota(jnp.int32, 16)
    out = jnp.zeros((16,), jnp.int32)
    for lane, val in pairs:
        out = jnp.where(io == lane, val, out)
    return out


@functools.partial(
    pl.kernel,
    out_type=(jax.ShapeDtypeStruct((_R, _C), jnp.int32),
              jax.ShapeDtypeStruct((_NW, 16), jnp.float32),
              jax.ShapeDtypeStruct((_NW, 16), jnp.int32),
              jax.ShapeDtypeStruct((_NW, _NB1), jnp.float32)),
    mesh=_MESH,
    scratch_types=[pltpu.VMEM((_C,), jnp.float32),
                   pltpu.VMEM((_C,), jnp.int32),
                   pltpu.VMEM((_C,), jnp.int32),
                   pltpu.VMEM((_NB1,), jnp.float32),
                   pltpu.VMEM((16,), jnp.float32),
                   pltpu.VMEM((16,), jnp.int32)],
    compiler_params=_CP,
)
def _c1(v_hbm, pm_hbm, nm_hbm, keys_hbm, posrow_hbm, counts_hbm, hist1_hbm,
        v_v, pm_v, nm_v, h_v, pr_v, ct_v):
    w = _wid()
    _zero_hist(h_v, _NB1)

    ones = jnp.ones((16,), jnp.float32)
    psums = []
    npos_t = jnp.int32(0)
    nnz_t = jnp.int32(0)
    for r in range(_RPW):
        row = w * _RPW + r
        pltpu.sync_copy(v_hbm.at[row], v_v)
        pltpu.sync_copy(pm_hbm.at[row], pm_v)
        pltpu.sync_copy(nm_hbm.at[row], nm_v)

        def body(i, carry):
            psum, npos, nnz = carry
            sl = pl.ds(i * 16, 16)
            v = v_v[sl]
            pmv = pm_v[sl]
            nmv = nm_v[sl]
            psum = psum + v * pmv.astype(jnp.float32)
            flat, key = _keys_from(v, nmv)
            nm_v[sl] = key
            nnz = nnz + (flat != 0.0).astype(jnp.int32)
            bin1 = jnp.right_shift(key, 21) + 1024
            plsc.addupdate_scatter(h_v, [bin1], ones)
            return psum, npos + pmv, nnz

        psum, nposv, nnzv = plsc.parallel_loop(
            0, _NCH, step=1, unroll=8,
            carry=(jnp.zeros((16,), jnp.float32), jnp.zeros((16,), jnp.int32),
                   jnp.zeros((16,), jnp.int32)))(body)
        pltpu.sync_copy(nm_v, keys_hbm.at[row])
        psums.append((r, jnp.sum(psum)))
        npos_t = npos_t + jnp.sum(nposv)
        nnz_t = nnz_t + jnp.sum(nnzv)

    io = lax.iota(jnp.int32, 16)
    prv = jnp.zeros((16,), jnp.float32)
    for r, s in psums:
        prv = jnp.where(io == r, s, prv)
    pr_v[...] = prv
    ct_v[...] = _lane_pack([(0, npos_t), (1, nnz_t)])
    pltpu.sync_copy(pr_v, posrow_hbm.at[w])
    pltpu.sync_copy(ct_v, counts_hbm.at[w])
    pltpu.sync_copy(h_v, hist1_hbm.at[w])


@functools.partial(
    pl.kernel,
    out_type=(jax.ShapeDtypeStruct((_NW, _H2W), jnp.float32),
              jax.ShapeDtypeStruct((_NW, 16), jnp.int32)),
    mesh=_MESH,
    scratch_types=[pltpu.VMEM((_NW, _NB1), jnp.float32),
                   pltpu.VMEM((_NB1,), jnp.float32),
                   pltpu.VMEM((_NW, 16), jnp.int32),
                   pltpu.VMEM((_C,), jnp.int32),
                   pltpu.VMEM((_H2W,), jnp.float32),
                   pltpu.VMEM((16,), jnp.int32)],
    compiler_params=_CP,
)
def _c2(counts_hbm, hist1_hbm, keys_hbm, hist2_hbm, sel2_hbm,
        hall_v, mrg_v, ct_v, k_v, h_v, sel_v):
    w = _wid()
    pltpu.sync_copy(hist1_hbm, hall_v)
    pltpu.sync_copy(counts_hbm, ct_v)

    _merge_hists(hall_v, mrg_v, _NB1)

    cts = _tree_sum([ct_v[w2, :] for w2 in range(_NW)])
    npos = cts[0]
    nnz = cts[1]
    k_tot = jnp.minimum(
        jnp.maximum(jnp.int32(_RATIO) * npos, jnp.int32(_MIN_NEG)), nnz)

    b1, above1 = _scan_bins(mrg_v, _NB1, k_tot)
    k_rem = k_tot - above1.astype(jnp.int32)

    _zero_hist(h_v, _H2W)

    ones = jnp.ones((16,), jnp.float32)
    for r in range(_RPW):
        row = w * _RPW + r
        pltpu.sync_copy(keys_hbm.at[row], k_v)

        def body(i):
            key = k_v[pl.ds(i * 16, 16)]
            match = (jnp.right_shift(key, 21) + 1024) == b1
            bin2 = jnp.right_shift(key, 10) & jnp.int32(0x7FF)
            idx = jnp.where(match, bin2, jnp.int32(_NB2))
            plsc.addupdate_scatter(h_v, [idx], ones)
        plsc.parallel_loop(0, _NCH, step=1, unroll=8)(body)

    sel_v[...] = _lane_pack([(0, b1), (1, k_rem), (3, k_tot)])
    pltpu.sync_copy(h_v, hist2_hbm.at[w])
    pltpu.sync_copy(sel_v, sel2_hbm.at[w])


@functools.partial(
    pl.kernel,
    out_type=(jax.ShapeDtypeStruct((_NW, _H3W), jnp.float32),
              jax.ShapeDtypeStruct((_NW, 16), jnp.int32)),
    mesh=_MESH,
    scratch_types=[pltpu.VMEM((_NW, _H2W), jnp.float32),
                   pltpu.VMEM((_NB2,), jnp.float32),
                   pltpu.VMEM((16,), jnp.int32),
                   pltpu.VMEM((_C,), jnp.int32),
                   pltpu.VMEM((_H3W,), jnp.float32),
                   pltpu.VMEM((16,), jnp.int32)],
    compiler_params=_CP,
)
def _c3(sel2_hbm, hist2_hbm, keys_hbm, hist3_hbm, sel3_hbm,
        hall_v, mrg_v, s_v, k_v, h_v, sel_v):
    w = _wid()
    pltpu.sync_copy(hist2_hbm, hall_v)
    pltpu.sync_copy(sel2_hbm.at[0], s_v)
    sel = s_v[...]
    b1 = sel[0]
    k_in = sel[1]
    k_tot = sel[3]

    _merge_hists(hall_v, mrg_v, _NB2)

    b2, above2 = _scan_bins(mrg_v, _NB2, k_in)
    k_rem = k_in - above2.astype(jnp.int32)
    top22s = jnp.left_shift(b1 - 1024, 11) + b2

    _zero_hist(h_v, _H3W)

    ones = jnp.ones((16,), jnp.float32)
    for r in range(_RPW):
        row = w * _RPW + r
        pltpu.sync_copy(keys_hbm.at[row], k_v)

        def body(i):
            key = k_v[pl.ds(i * 16, 16)]
            match = jnp.right_shift(key, 10) == top22s
            bin3 = key & jnp.int32(0x3FF)
            idx = jnp.where(match, bin3, jnp.int32(_NB3))
            plsc.addupdate_scatter(h_v, [idx], ones)
        plsc.parallel_loop(0, _NCH, step=1, unroll=8)(body)

    sel_v[...] = _lane_pack([(0, top22s), (1, k_rem), (3, k_tot)])
    pltpu.sync_copy(h_v, hist3_hbm.at[w])
    pltpu.sync_copy(sel_v, sel3_hbm.at[w])


@functools.partial(
    pl.kernel,
    out_type=(jax.ShapeDtypeStruct((_NW, 16), jnp.int32),
              jax.ShapeDtypeStruct((_NW, 16), jnp.int32)),
    mesh=_MESH,
    scratch_types=[pltpu.VMEM((_NW, _H3W), jnp.float32),
                   pltpu.VMEM((_NB3,), jnp.float32),
                   pltpu.VMEM((16,), jnp.int32),
                   pltpu.VMEM((_C,), jnp.int32),
                   pltpu.VMEM((16,), jnp.int32),
                   pltpu.VMEM((16,), jnp.int32)],
    compiler_params=_CP,
)
def _c4(sel3_hbm, hist3_hbm, keys_hbm, rowties_hbm, sel4_hbm,
        hall_v, mrg_v, s_v, k_v, rt_v, sel_v):
    w = _wid()
    pltpu.sync_copy(hist3_hbm, hall_v)
    pltpu.sync_copy(sel3_hbm.at[0], s_v)
    sel = s_v[...]
    top22s = sel[0]
    k_in = sel[1]
    k_tot = sel[3]

    _merge_hists(hall_v, mrg_v, _NB3)

    b3, above3 = _scan_bins(mrg_v, _NB3, k_in)
    c_take = k_in - above3.astype(jnp.int32)
    t_key = jnp.left_shift(top22s, 10) | b3

    ties = []
    for r in range(_RPW):
        row = w * _RPW + r
        pltpu.sync_copy(keys_hbm.at[row], k_v)

        def body(i, acc):
            key = k_v[pl.ds(i * 16, 16)]
            return acc + (key == t_key).astype(jnp.int32)
        tv = plsc.parallel_loop(
            0, _NCH, step=1, unroll=8,
            carry=jnp.zeros((16,), jnp.int32))(body)
        ties.append((r, jnp.sum(tv)))

    rt_v[...] = _lane_pack(ties)
    sel_v[...] = _lane_pack([(0, t_key), (1, c_take), (3, k_tot)])
    pltpu.sync_copy(rt_v, rowties_hbm.at[w])
    pltpu.sync_copy(sel_v, sel4_hbm.at[w])


@functools.partial(
    pl.kernel,
    out_type=jax.ShapeDtypeStruct((_NW, 16), jnp.float32),
    mesh=_MESH,
    scratch_types=[pltpu.VMEM((_NW, 16), jnp.int32),
                   pltpu.VMEM((16,), jnp.int32),
                   pltpu.VMEM((16,), jnp.float32),
                   pltpu.VMEM((_C,), jnp.int32),
                   pltpu.VMEM((_C,), jnp.float32),
                   pltpu.VMEM((16,), jnp.float32)],
    compiler_params=_CP,
)
def _c5(sel4_hbm, rowties_hbm, posrow_hbm, keys_hbm, v_hbm, out_hbm,
        rt_v, s_v, pr_v, k_v, v_v, o_v):
    w = _wid()
    pltpu.sync_copy(rowties_hbm, rt_v)
    pltpu.sync_copy(sel4_hbm.at[0], s_v)
    pltpu.sync_copy(posrow_hbm.at[w], pr_v)
    sel = s_v[...]
    t_key = sel[0]
    c_take = sel[1]

    def pre(w2, acc):
        return acc + rt_v[w2, :]
    prev = lax.fori_loop(0, w, pre, jnp.zeros((16,), jnp.int32))
    excl0 = prev[0] + prev[1]
    own = rt_v[w, :]
    rt0 = own[0]
    rt1 = own[1]
    take0 = jnp.clip(c_take - excl0, 0, rt0)
    take1 = jnp.clip(c_take - (excl0 + rt0), 0, rt1)
    takes = (take0, take1)
    pr = pr_v[...]

    outs = []
    for r in range(_RPW):
        row = w * _RPW + r
        pltpu.sync_copy(keys_hbm.at[row], k_v)
        pltpu.sync_copy(v_hbm.at[row], v_v)
        take_r = takes[r]

        def body(i, carry):
            negacc, run = carry
            for u in range(5):
                sl = pl.ds(i * 80 + u * 16, 16)
                key = k_v[sl]
                v = v_v[sl]
                tie = key == t_key
                tiei = tie.astype(jnp.int32)
                incl = plsc.cumsum(tiei)
                rank = run + incl - tiei
                keep = (key > t_key) | (tie & (rank < take_r))
                negacc = negacc + jnp.where(keep, v, 0.0)
                run = run + incl[15]
            return negacc, run

        negacc, _ = lax.fori_loop(
            0, _NCH // 5, body, (jnp.zeros((16,), jnp.float32), jnp.int32(0)))
        outs.append((r, pr[r] + jnp.sum(negacc)))

    io = lax.iota(jnp.int32, 16)
    ov = jnp.zeros((16,), jnp.float32)
    for r, s in outs:
        ov = jnp.where(io == r, s, ov)
    o_v[...] = ov
    pltpu.sync_copy(o_v, out_hbm.at[w])


def kernel(values, positive_mask, negative_mask):
    keys, posrow, counts, hist1 = _c1(values, positive_mask, negative_mask)
    hist2, sel2 = _c2(counts, hist1, keys)
    hist3, sel3 = _c3(sel2, hist2, keys)
    rowties, sel4 = _c4(sel3, hist3, keys)
    out = _c5(sel4, rowties, posrow, keys, values)
    return out[:, :_RPW].reshape(_R)


# speedup vs baseline: 1.2310x; 1.0141x over previous
import functools

import jax
import jax.numpy as jnp
from jax import lax
from jax.experimental import pallas as pl
from jax.experimental.pallas import tpu as pltpu
from jax.experimental.pallas import tpu_sc as plsc

_RATIO = 3
_MIN_NEG = 0
_R = 64
_C = 20000
_NW = 32
_RPW = _R // _NW
_NCH = _C // 16
_INT_MIN = -2147483648

_MESH = plsc.VectorSubcoreMesh(core_axis_name="c", subcore_axis_name="s")
_CP = pltpu.CompilerParams(needs_layout_passes=False)

_NB1 = 2048
_NB2 = 2048
_NB3 = 1024
_H2W = _NB2 + 16
_H3W = _NB3 + 16


def _wid():
    return lax.axis_index("s") * 2 + lax.axis_index("c")


def _tree_sum(vals):
    while len(vals) > 1:
        vals = [a + b for a, b in zip(vals[::2], vals[1::2])]
    return vals[0]


def _merge_hists(hall_v, mrg_v, nbins):
    def merge(j):
        sl = pl.ds(j * 16, 16)
        mrg_v[sl] = _tree_sum([hall_v[w2, sl] for w2 in range(_NW)])
    plsc.parallel_loop(0, nbins // 16, step=1, unroll=2)(merge)


def _zero_hist(h_v, width):
    def zh(i):
        h_v[pl.ds(i * 16, 16)] = jnp.zeros((16,), jnp.float32)
    plsc.parallel_loop(0, width // 16, step=1, unroll=8)(zh)


def _keys_from(v, nmv):
    flat = v * nmv.astype(jnp.float32)
    b = plsc.bitcast(flat, jnp.int32)
    key = jnp.where(b >= 0, b, b ^ jnp.int32(0x7FFFFFFF))
    return flat, key


def _scan_bins(merged_ref, nbins, k_rem):
    nch = nbins // 16
    kf = k_rem.astype(jnp.float32)

    def body(t, carry):
        acc, found, bsel, above = carry
        base = nbins - 16 * (t + 1)
        h = merged_ref[pl.ds(base, 16)]
        cs = plsc.cumsum(h)
        chunk_sum = cs[15]
        suff = acc + (chunk_sum - cs) + h
        m = suff >= kf
        pc = plsc.all_reduce_population_count(m)[0]
        has = pc > 0
        lane = pc - 1
        csl = jnp.max(jnp.where(m, cs, 0.0))
        hit = jnp.logical_and(has, jnp.logical_not(found))
        bsel = jnp.where(hit, base + lane, bsel)
        above = jnp.where(hit, acc + chunk_sum - csl, above)
        found = jnp.logical_or(found, has)
        return acc + chunk_sum, found, bsel, above

    _, _, bsel, above = lax.fori_loop(
        0, nch, body,
        (jnp.float32(0.0), jnp.bool_(False), jnp.int32(0), jnp.float32(0.0)))
    return bsel, above


def _lane_pack(pairs):
    io = lax.iota(jnp.int32, 16)
    out = jnp.zeros((16,), jnp.int32)
    for lane, val in pairs:
        out = jnp.where(io == lane, val, out)
    return out


@functools.partial(
    pl.kernel,
    out_type=(jax.ShapeDtypeStruct((_R, _C), jnp.int32),
              jax.ShapeDtypeStruct((_NW, 16), jnp.float32),
              jax.ShapeDtypeStruct((_NW, 16), jnp.int32),
              jax.ShapeDtypeStruct((_NW, _NB1), jnp.float32)),
    mesh=_MESH,
    scratch_types=[pltpu.VMEM((_C,), jnp.float32),
                   pltpu.VMEM((_C,), jnp.int32),
                   pltpu.VMEM((_C,), jnp.int32),
                   pltpu.VMEM((_NB1,), jnp.float32),
                   pltpu.VMEM((16,), jnp.float32),
                   pltpu.VMEM((16,), jnp.int32)],
    compiler_params=_CP,
)
def _c1(v_hbm, pm_hbm, nm_hbm, keys_hbm, posrow_hbm, counts_hbm, hist1_hbm,
        v_v, pm_v, nm_v, h_v, pr_v, ct_v):
    w = _wid()
    _zero_hist(h_v, _NB1)

    ones = jnp.ones((16,), jnp.float32)
    psums = []
    npos_t = jnp.int32(0)
    nnz_t = jnp.int32(0)
    for r in range(_RPW):
        row = w * _RPW + r
        pltpu.sync_copy(v_hbm.at[row], v_v)
        pltpu.sync_copy(pm_hbm.at[row], pm_v)
        pltpu.sync_copy(nm_hbm.at[row], nm_v)

        def body(i, carry):
            psum, npos, nnz = carry
            sl = pl.ds(i * 16, 16)
            v = v_v[sl]
            pmv = pm_v[sl]
            nmv = nm_v[sl]
            psum = psum + v * pmv.astype(jnp.float32)
            flat, key = _keys_from(v, nmv)
            nm_v[sl] = key
            nnz = nnz + (flat != 0.0).astype(jnp.int32)
            bin1 = jnp.right_shift(key, 21) + 1024
            plsc.addupdate_scatter(h_v, [bin1], ones)
            return psum, npos + pmv, nnz

        psum, nposv, nnzv = plsc.parallel_loop(
            0, _NCH, step=1, unroll=8,
            carry=(jnp.zeros((16,), jnp.float32), jnp.zeros((16,), jnp.int32),
                   jnp.zeros((16,), jnp.int32)))(body)
        pltpu.sync_copy(nm_v, keys_hbm.at[row])
        psums.append((r, jnp.sum(psum)))
        npos_t = npos_t + jnp.sum(nposv)
        nnz_t = nnz_t + jnp.sum(nnzv)

    io = lax.iota(jnp.int32, 16)
    prv = jnp.zeros((16,), jnp.float32)
    for r, s in psums:
        prv = jnp.where(io == r, s, prv)
    pr_v[...] = prv
    ct_v[...] = _lane_pack([(0, npos_t), (1, nnz_t)])
    pltpu.sync_copy(pr_v, posrow_hbm.at[w])
    pltpu.sync_copy(ct_v, counts_hbm.at[w])
    pltpu.sync_copy(h_v, hist1_hbm.at[w])


@functools.partial(
    pl.kernel,
    out_type=(jax.ShapeDtypeStruct((_NW, _H2W), jnp.float32),
              jax.ShapeDtypeStruct((_NW, 16), jnp.int32)),
    mesh=_MESH,
    scratch_types=[pltpu.VMEM((_NW, _NB1), jnp.float32),
                   pltpu.VMEM((_NB1,), jnp.float32),
                   pltpu.VMEM((_NW, 16), jnp.int32),
                   pltpu.VMEM((_C,), jnp.int32),
                   pltpu.VMEM((_H2W,), jnp.float32),
                   pltpu.VMEM((16,), jnp.int32)],
    compiler_params=_CP,
)
def _c2(counts_hbm, hist1_hbm, keys_hbm, hist2_hbm, sel2_hbm,
        hall_v, mrg_v, ct_v, k_v, h_v, sel_v):
    w = _wid()
    pltpu.sync_copy(hist1_hbm, hall_v)
    pltpu.sync_copy(counts_hbm, ct_v)

    _merge_hists(hall_v, mrg_v, _NB1)

    cts = _tree_sum([ct_v[w2, :] for w2 in range(_NW)])
    npos = cts[0]
    nnz = cts[1]
    k_tot = jnp.minimum(
        jnp.maximum(jnp.int32(_RATIO) * npos, jnp.int32(_MIN_NEG)), nnz)

    b1, above1 = _scan_bins(mrg_v, _NB1, k_tot)
    k_rem = k_tot - above1.astype(jnp.int32)

    _zero_hist(h_v, _H2W)

    ones = jnp.ones((16,), jnp.float32)
    for r in range(_RPW):
        row = w * _RPW + r
        pltpu.sync_copy(keys_hbm.at[row], k_v)

        def body(i):
            key = k_v[pl.ds(i * 16, 16)]
            match = (jnp.right_shift(key, 21) + 1024) == b1
            bin2 = jnp.right_shift(key, 10) & jnp.int32(0x7FF)
            idx = jnp.where(match, bin2, jnp.int32(_NB2))
            plsc.addupdate_scatter(h_v, [idx], ones)
        plsc.parallel_loop(0, _NCH, step=1, unroll=8)(body)

    sel_v[...] = _lane_pack([(0, b1), (1, k_rem), (3, k_tot)])
    pltpu.sync_copy(h_v, hist2_hbm.at[w])
    pltpu.sync_copy(sel_v, sel2_hbm.at[w])


@functools.partial(
    pl.kernel,
    out_type=(jax.ShapeDtypeStruct((_NW, _H3W), jnp.float32),
              jax.ShapeDtypeStruct((_NW, 16), jnp.int32)),
    mesh=_MESH,
    scratch_types=[pltpu.VMEM((_NW, _H2W), jnp.float32),
                   pltpu.VMEM((_NB2,), jnp.float32),
                   pltpu.VMEM((16,), jnp.int32),
                   pltpu.VMEM((_C,), jnp.int32),
                   pltpu.VMEM((_H3W,), jnp.float32),
                   pltpu.VMEM((16,), jnp.int32)],
    compiler_params=_CP,
)
def _c3(sel2_hbm, hist2_hbm, keys_hbm, hist3_hbm, sel3_hbm,
        hall_v, mrg_v, s_v, k_v, h_v, sel_v):
    w = _wid()
    pltpu.sync_copy(hist2_hbm, hall_v)
    pltpu.sync_copy(sel2_hbm.at[0], s_v)
    sel = s_v[...]
    b1 = sel[0]
    k_in = sel[1]
    k_tot = sel[3]

    _merge_hists(hall_v, mrg_v, _NB2)

    b2, above2 = _scan_bins(mrg_v, _NB2, k_in)
    k_rem = k_in - above2.astype(jnp.int32)
    top22s = jnp.left_shift(b1 - 1024, 11) + b2

    _zero_hist(h_v, _H3W)

    ones = jnp.ones((16,), jnp.float32)
    for r in range(_RPW):
        row = w * _RPW + r
        pltpu.sync_copy(keys_hbm.at[row], k_v)

        def body(i):
            key = k_v[pl.ds(i * 16, 16)]
            match = jnp.right_shift(key, 10) == top22s
            bin3 = key & jnp.int32(0x3FF)
            idx = jnp.where(match, bin3, jnp.int32(_NB3))
            plsc.addupdate_scatter(h_v, [idx], ones)
        plsc.parallel_loop(0, _NCH, step=1, unroll=8)(body)

    sel_v[...] = _lane_pack([(0, top22s), (1, k_rem), (3, k_tot)])
    pltpu.sync_copy(h_v, hist3_hbm.at[w])
    pltpu.sync_copy(sel_v, sel3_hbm.at[w])


@functools.partial(
    pl.kernel,
    out_type=(jax.ShapeDtypeStruct((_NW, 16), jnp.int32),
              jax.ShapeDtypeStruct((_NW, 16), jnp.int32)),
    mesh=_MESH,
    scratch_types=[pltpu.VMEM((_NW, _H3W), jnp.float32),
                   pltpu.VMEM((_NB3,), jnp.float32),
                   pltpu.VMEM((16,), jnp.int32),
                   pltpu.VMEM((_C,), jnp.int32),
                   pltpu.VMEM((16,), jnp.int32),
                   pltpu.VMEM((16,), jnp.int32)],
    compiler_params=_CP,
)
def _c4(sel3_hbm, hist3_hbm, keys_hbm, rowties_hbm, sel4_hbm,
        hall_v, mrg_v, s_v, k_v, rt_v, sel_v):
    w = _wid()
    pltpu.sync_copy(hist3_hbm, hall_v)
    pltpu.sync_copy(sel3_hbm.at[0], s_v)
    sel = s_v[...]
    top22s = sel[0]
    k_in = sel[1]
    k_tot = sel[3]

    _merge_hists(hall_v, mrg_v, _NB3)

    b3, above3 = _scan_bins(mrg_v, _NB3, k_in)
    c_take = k_in - above3.astype(jnp.int32)
    t_key = jnp.left_shift(top22s, 10) | b3

    ties = []
    for r in range(_RPW):
        row = w * _RPW + r
        pltpu.sync_copy(keys_hbm.at[row], k_v)

        def body(i, acc):
            key = k_v[pl.ds(i * 16, 16)]
            return acc + (key == t_key).astype(jnp.int32)
        tv = plsc.parallel_loop(
            0, _NCH, step=1, unroll=8,
            carry=jnp.zeros((16,), jnp.int32))(body)
        ties.append((r, jnp.sum(tv)))

    rt_v[...] = _lane_pack(ties)
    sel_v[...] = _lane_pack([(0, t_key), (1, c_take), (3, k_tot)])
    pltpu.sync_copy(rt_v, rowties_hbm.at[w])
    pltpu.sync_copy(sel_v, sel4_hbm.at[w])


@functools.partial(
    pl.kernel,
    out_type=jax.ShapeDtypeStruct((_NW, 16), jnp.float32),
    mesh=_MESH,
    scratch_types=[pltpu.VMEM((_NW, 16), jnp.int32),
                   pltpu.VMEM((16,), jnp.int32),
                   pltpu.VMEM((16,), jnp.float32),
                   pltpu.VMEM((_C,), jnp.int32),
                   pltpu.VMEM((_C,), jnp.float32),
                   pltpu.VMEM((16,), jnp.float32)],
    compiler_params=_CP,
)
def _c5(sel4_hbm, rowties_hbm, posrow_hbm, keys_hbm, v_hbm, out_hbm,
        rt_v, s_v, pr_v, k_v, v_v, o_v):
    w = _wid()
    pltpu.sync_copy(rowties_hbm, rt_v)
    pltpu.sync_copy(sel4_hbm.at[0], s_v)
    pltpu.sync_copy(posrow_hbm.at[w], pr_v)
    sel = s_v[...]
    t_key = sel[0]
    c_take = sel[1]

    def pre(w2, acc):
        return acc + rt_v[w2, :]
    prev = lax.fori_loop(0, w, pre, jnp.zeros((16,), jnp.int32))
    excl0 = prev[0] + prev[1]
    own = rt_v[w, :]
    rt0 = own[0]
    rt1 = own[1]
    take0 = jnp.clip(c_take - excl0, 0, rt0)
    take1 = jnp.clip(c_take - (excl0 + rt0), 0, rt1)
    takes = (take0, take1)
    pr = pr_v[...]

    outs = []
    for r in range(_RPW):
        row = w * _RPW + r
        pltpu.sync_copy(keys_hbm.at[row], k_v)
        pltpu.sync_copy(v_hbm.at[row], v_v)
        take_r = takes[r]

        def body(i, carry):
            negacc, run = carry
            for u in range(5):
                sl = pl.ds(i * 80 + u * 16, 16)
                key = k_v[sl]
                v = v_v[sl]
                tie = key == t_key
                tiei = tie.astype(jnp.int32)
                incl = plsc.cumsum(tiei)
                rank = run + incl - tiei
                keep = (key > t_key) | (tie & (rank < take_r))
                negacc = negacc + jnp.where(keep, v, 0.0)
                run = run + incl[15]
            return negacc, run

        negacc, _ = lax.fori_loop(
            0, _NCH // 5, body, (jnp.zeros((16,), jnp.float32), jnp.int32(0)))
        outs.append((r, pr[r] + jnp.sum(negacc)))

    io = lax.iota(jnp.int32, 16)
    ov = jnp.zeros((16,), jnp.float32)
    for r, s in outs:
        ov = jnp.where(io == r, s, ov)
    o_v[...] = ov
    pltpu.sync_copy(o_v, out_hbm.at[w])


def kernel(values, positive_mask, negative_mask):
    keys, posrow, counts, hist1 = _c1(values, positive_mask, negative_mask)
    hist2, sel2 = _c2(counts, hist1, keys)
    hist3, sel3 = _c3(sel2, hist2, keys)
    rowties, sel4 = _c4(sel3, hist3, keys)
    out = _c5(sel4, rowties, posrow, keys, values)
    return out[:, :_RPW].reshape(_R)


# speedup vs baseline: 1.2449x; 1.0113x over previous
import functools

import jax
import jax.numpy as jnp
from jax import lax
from jax.experimental import pallas as pl
from jax.experimental.pallas import tpu as pltpu
from jax.experimental.pallas import tpu_sc as plsc

_RATIO = 3
_MIN_NEG = 0
_R = 64
_C = 20000
_NW = 32
_RPW = _R // _NW
_NCH = _C // 16
_INT_MIN = -2147483648

_MESH = plsc.VectorSubcoreMesh(core_axis_name="c", subcore_axis_name="s")
_CP = pltpu.CompilerParams(needs_layout_passes=False)

_NB1 = 2048
_NB2 = 2048
_NB3 = 1024
_H2W = _NB2 + 16
_H3W = _NB3 + 16


def _wid():
    return lax.axis_index("s") * 2 + lax.axis_index("c")


def _tree_sum(vals):
    while len(vals) > 1:
        vals = [a + b for a, b in zip(vals[::2], vals[1::2])]
    return vals[0]


def _merge_hists(hall_v, mrg_v, nbins):
    def merge(j):
        sl = pl.ds(j * 16, 16)
        mrg_v[sl] = _tree_sum([hall_v[w2, sl] for w2 in range(_NW)])
    plsc.parallel_loop(0, nbins // 16, step=1, unroll=2)(merge)


def _zero_hist(h_v, width):
    def zh(i):
        h_v[pl.ds(i * 16, 16)] = jnp.zeros((16,), jnp.float32)
    plsc.parallel_loop(0, width // 16, step=1, unroll=8)(zh)


def _keys_from(v, nmv):
    flat = v * nmv.astype(jnp.float32)
    b = plsc.bitcast(flat, jnp.int32)
    key = jnp.where(b >= 0, b, b ^ jnp.int32(0x7FFFFFFF))
    return flat, key


def _scan_bins(merged_ref, nbins, k_rem):
    nch = nbins // 16
    kf = k_rem.astype(jnp.float32)

    def body(t, carry):
        acc, found, bsel, above = carry
        base = nbins - 16 * (t + 1)
        h = merged_ref[pl.ds(base, 16)]
        cs = plsc.cumsum(h)
        chunk_sum = cs[15]
        suff = acc + (chunk_sum - cs) + h
        m = suff >= kf
        pc = plsc.all_reduce_population_count(m)[0]
        has = pc > 0
        lane = pc - 1
        csl = jnp.max(jnp.where(m, cs, 0.0))
        hit = jnp.logical_and(has, jnp.logical_not(found))
        bsel = jnp.where(hit, base + lane, bsel)
        above = jnp.where(hit, acc + chunk_sum - csl, above)
        found = jnp.logical_or(found, has)
        return acc + chunk_sum, found, bsel, above

    _, _, bsel, above = lax.fori_loop(
        0, nch, body,
        (jnp.float32(0.0), jnp.bool_(False), jnp.int32(0), jnp.float32(0.0)))
    return bsel, above


def _lane_pack(pairs):
    io = lax.iota(jnp.int32, 16)
    out = jnp.zeros((16,), jnp.int32)
    for lane, val in pairs:
        out = jnp.where(io == lane, val, out)
    return out


@functools.partial(
    pl.kernel,
    out_type=(jax.ShapeDtypeStruct((_R, _C), jnp.int32),
              jax.ShapeDtypeStruct((_NW, 16), jnp.float32),
              jax.ShapeDtypeStruct((_NW, 16), jnp.int32),
              jax.ShapeDtypeStruct((_NW, _NB1), jnp.float32)),
    mesh=_MESH,
    scratch_types=[pltpu.VMEM((_C,), jnp.float32),
                   pltpu.VMEM((_C,), jnp.int32),
                   pltpu.VMEM((_C,), jnp.int32),
                   pltpu.VMEM((_NB1,), jnp.float32),
                   pltpu.VMEM((16,), jnp.float32),
                   pltpu.VMEM((16,), jnp.int32)],
    compiler_params=_CP,
)
def _c1(v_hbm, pm_hbm, nm_hbm, keys_hbm, posrow_hbm, counts_hbm, hist1_hbm,
        v_v, pm_v, nm_v, h_v, pr_v, ct_v):
    w = _wid()
    _zero_hist(h_v, _NB1)

    ones = jnp.ones((16,), jnp.float32)
    psums = []
    npos_t = jnp.int32(0)
    nnz_t = jnp.int32(0)
    for r in range(_RPW):
        row = w * _RPW + r
        pltpu.sync_copy(v_hbm.at[row], v_v)
        pltpu.sync_copy(pm_hbm.at[row], pm_v)
        pltpu.sync_copy(nm_hbm.at[row], nm_v)

        def body(i, carry):
            psum, npos, nnz = carry
            sl = pl.ds(i * 16, 16)
            v = v_v[sl]
            pmv = pm_v[sl]
            nmv = nm_v[sl]
            psum = psum + v * pmv.astype(jnp.float32)
            flat, key = _keys_from(v, nmv)
            nm_v[sl] = key
            nnz = nnz + (flat != 0.0).astype(jnp.int32)
            bin1 = jnp.right_shift(key, 21) + 1024
            plsc.addupdate_scatter(h_v, [bin1], ones)
            return psum, npos + pmv, nnz

        psum, nposv, nnzv = plsc.parallel_loop(
            0, _NCH, step=1, unroll=8,
            carry=(jnp.zeros((16,), jnp.float32), jnp.zeros((16,), jnp.int32),
                   jnp.zeros((16,), jnp.int32)))(body)
        pltpu.sync_copy(nm_v, keys_hbm.at[row])
        psums.append((r, jnp.sum(psum)))
        npos_t = npos_t + jnp.sum(nposv)
        nnz_t = nnz_t + jnp.sum(nnzv)

    io = lax.iota(jnp.int32, 16)
    prv = jnp.zeros((16,), jnp.float32)
    for r, s in psums:
        prv = jnp.where(io == r, s, prv)
    pr_v[...] = prv
    ct_v[...] = _lane_pack([(0, npos_t), (1, nnz_t)])
    pltpu.sync_copy(pr_v, posrow_hbm.at[w])
    pltpu.sync_copy(ct_v, counts_hbm.at[w])
    pltpu.sync_copy(h_v, hist1_hbm.at[w])


@functools.partial(
    pl.kernel,
    out_type=(jax.ShapeDtypeStruct((_NW, _H2W), jnp.float32),
              jax.ShapeDtypeStruct((_NW, 16), jnp.int32)),
    mesh=_MESH,
    scratch_types=[pltpu.VMEM((_NW, _NB1), jnp.float32),
                   pltpu.VMEM((_NB1,), jnp.float32),
                   pltpu.VMEM((_NW, 16), jnp.int32),
                   pltpu.VMEM((_C,), jnp.int32),
                   pltpu.VMEM((_H2W,), jnp.float32),
                   pltpu.VMEM((16,), jnp.int32)],
    compiler_params=_CP,
)
def _c2(counts_hbm, hist1_hbm, keys_hbm, hist2_hbm, sel2_hbm,
        hall_v, mrg_v, ct_v, k_v, h_v, sel_v):
    w = _wid()
    pltpu.sync_copy(hist1_hbm, hall_v)
    pltpu.sync_copy(counts_hbm, ct_v)

    _merge_hists(hall_v, mrg_v, _NB1)

    cts = _tree_sum([ct_v[w2, :] for w2 in range(_NW)])
    npos = cts[0]
    nnz = cts[1]
    k_tot = jnp.minimum(
        jnp.maximum(jnp.int32(_RATIO) * npos, jnp.int32(_MIN_NEG)), nnz)

    b1, above1 = _scan_bins(mrg_v, _NB1, k_tot)
    k_rem = k_tot - above1.astype(jnp.int32)

    _zero_hist(h_v, _H2W)

    ones = jnp.ones((16,), jnp.float32)
    for r in range(_RPW):
        row = w * _RPW + r
        pltpu.sync_copy(keys_hbm.at[row], k_v)

        def body(i):
            key = k_v[pl.ds(i * 16, 16)]
            match = (jnp.right_shift(key, 21) + 1024) == b1
            bin2 = jnp.right_shift(key, 10) & jnp.int32(0x7FF)
            idx = jnp.where(match, bin2, jnp.int32(_NB2))
            plsc.addupdate_scatter(h_v, [idx], ones)
        plsc.parallel_loop(0, _NCH, step=1, unroll=8)(body)

    sel_v[...] = _lane_pack([(0, b1), (1, k_rem), (3, k_tot)])
    pltpu.sync_copy(h_v, hist2_hbm.at[w])
    pltpu.sync_copy(sel_v, sel2_hbm.at[w])


@functools.partial(
    pl.kernel,
    out_type=(jax.ShapeDtypeStruct((_NW, _H3W), jnp.float32),
              jax.ShapeDtypeStruct((_NW, 16), jnp.int32)),
    mesh=_MESH,
    scratch_types=[pltpu.VMEM((_NW, _H2W), jnp.float32),
                   pltpu.VMEM((_NB2,), jnp.float32),
                   pltpu.VMEM((16,), jnp.int32),
                   pltpu.VMEM((_C,), jnp.int32),
                   pltpu.VMEM((_H3W,), jnp.float32),
                   pltpu.VMEM((16,), jnp.int32)],
    compiler_params=_CP,
)
def _c3(sel2_hbm, hist2_hbm, keys_hbm, hist3_hbm, sel3_hbm,
        hall_v, mrg_v, s_v, k_v, h_v, sel_v):
    w = _wid()
    pltpu.sync_copy(hist2_hbm, hall_v)
    pltpu.sync_copy(sel2_hbm.at[0], s_v)
    sel = s_v[...]
    b1 = sel[0]
    k_in = sel[1]
    k_tot = sel[3]

    _merge_hists(hall_v, mrg_v, _NB2)

    b2, above2 = _scan_bins(mrg_v, _NB2, k_in)
    k_rem = k_in - above2.astype(jnp.int32)
    top22s = jnp.left_shift(b1 - 1024, 11) + b2

    _zero_hist(h_v, _H3W)

    ones = jnp.ones((16,), jnp.float32)
    for r in range(_RPW):
        row = w * _RPW + r
        pltpu.sync_copy(keys_hbm.at[row], k_v)

        def body(i):
            key = k_v[pl.ds(i * 16, 16)]
            match = jnp.right_shift(key, 10) == top22s
            bin3 = key & jnp.int32(0x3FF)
            idx = jnp.where(match, bin3, jnp.int32(_NB3))
            plsc.addupdate_scatter(h_v, [idx], ones)
        plsc.parallel_loop(0, _NCH, step=1, unroll=8)(body)

    sel_v[...] = _lane_pack([(0, top22s), (1, k_rem), (3, k_tot)])
    pltpu.sync_copy(h_v, hist3_hbm.at[w])
    pltpu.sync_copy(sel_v, sel3_hbm.at[w])


@functools.partial(
    pl.kernel,
    out_type=(jax.ShapeDtypeStruct((_NW, 16), jnp.int32),
              jax.ShapeDtypeStruct((_NW, 16), jnp.int32)),
    mesh=_MESH,
    scratch_types=[pltpu.VMEM((_NW, _H3W), jnp.float32),
                   pltpu.VMEM((_NB3,), jnp.float32),
                   pltpu.VMEM((16,), jnp.int32),
                   pltpu.VMEM((_C,), jnp.int32),
                   pltpu.VMEM((16,), jnp.int32),
                   pltpu.VMEM((16,), jnp.int32)],
    compiler_params=_CP,
)
def _c4(sel3_hbm, hist3_hbm, keys_hbm, rowties_hbm, sel4_hbm,
        hall_v, mrg_v, s_v, k_v, rt_v, sel_v):
    w = _wid()
    pltpu.sync_copy(hist3_hbm, hall_v)
    pltpu.sync_copy(sel3_hbm.at[0], s_v)
    sel = s_v[...]
    top22s = sel[0]
    k_in = sel[1]
    k_tot = sel[3]

    _merge_hists(hall_v, mrg_v, _NB3)

    b3, above3 = _scan_bins(mrg_v, _NB3, k_in)
    c_take = k_in - above3.astype(jnp.int32)
    t_key = jnp.left_shift(top22s, 10) | b3

    ties = []
    for r in range(_RPW):
        row = w * _RPW + r
        pltpu.sync_copy(keys_hbm.at[row], k_v)

        def body(i, acc):
            key = k_v[pl.ds(i * 16, 16)]
            return acc + (key == t_key).astype(jnp.int32)
        tv = plsc.parallel_loop(
            0, _NCH, step=1, unroll=8,
            carry=jnp.zeros((16,), jnp.int32))(body)
        ties.append((r, jnp.sum(tv)))

    rt_v[...] = _lane_pack(ties)
    sel_v[...] = _lane_pack([(0, t_key), (1, c_take), (3, k_tot)])
    pltpu.sync_copy(rt_v, rowties_hbm.at[w])
    pltpu.sync_copy(sel_v, sel4_hbm.at[w])


@functools.partial(
    pl.kernel,
    out_type=jax.ShapeDtypeStruct((_NW, 16), jnp.float32),
    mesh=_MESH,
    scratch_types=[pltpu.VMEM((_NW, 16), jnp.int32),
                   pltpu.VMEM((16,), jnp.int32),
                   pltpu.VMEM((16,), jnp.float32),
                   pltpu.VMEM((_C,), jnp.int32),
                   pltpu.VMEM((_C,), jnp.float32),
                   pltpu.VMEM((16,), jnp.float32)],
    compiler_params=_CP,
)
def _c5(sel4_hbm, rowties_hbm, posrow_hbm, keys_hbm, v_hbm, out_hbm,
        rt_v, s_v, pr_v, k_v, v_v, o_v):
    w = _wid()
    pltpu.sync_copy(rowties_hbm, rt_v)
    pltpu.sync_copy(sel4_hbm.at[0], s_v)
    pltpu.sync_copy(posrow_hbm.at[w], pr_v)
    sel = s_v[...]
    t_key = sel[0]
    c_take = sel[1]

    def pre(w2, acc):
        return acc + rt_v[w2, :]
    prev = lax.fori_loop(0, w, pre, jnp.zeros((16,), jnp.int32))
    excl0 = prev[0] + prev[1]
    own = rt_v[w, :]
    rt0 = own[0]
    rt1 = own[1]
    take0 = jnp.clip(c_take - excl0, 0, rt0)
    take1 = jnp.clip(c_take - (excl0 + rt0), 0, rt1)
    takes = (take0, take1)
    pr = pr_v[...]

    outs = []
    for r in range(_RPW):
        row = w * _RPW + r
        pltpu.sync_copy(keys_hbm.at[row], k_v)
        pltpu.sync_copy(v_hbm.at[row], v_v)
        take_r = takes[r]
        rt_r = (rt0, rt1)[r]
        straddles = jnp.logical_and(take_r > 0, take_r < rt_r)

        def slow(_):
            def body(i, carry):
                negacc, run = carry
                sl = pl.ds(i * 16, 16)
                key = k_v[sl]
                v = v_v[sl]
                tie = key == t_key
                tiei = tie.astype(jnp.int32)
                incl = plsc.cumsum(tiei)
                rank = run + incl - tiei
                keep = (key > t_key) | (tie & (rank < take_r))
                negacc = negacc + jnp.where(keep, v, 0.0)
                return negacc, run + incl[15]
            negacc, _ = lax.fori_loop(
                0, _NCH, body, (jnp.zeros((16,), jnp.float32), jnp.int32(0)))
            return negacc

        def fast(_):
            all_ties = take_r > 0

            def body(i, acc):
                sl = pl.ds(i * 16, 16)
                key = k_v[sl]
                v = v_v[sl]
                keep = (key > t_key) | (all_ties & (key == t_key))
                return acc + jnp.where(keep, v, 0.0)
            return plsc.parallel_loop(
                0, _NCH, step=1, unroll=8,
                carry=jnp.zeros((16,), jnp.float32))(body)

        negacc = lax.cond(straddles, slow, fast, 0)
        outs.append((r, pr[r] + jnp.sum(negacc)))

    io = lax.iota(jnp.int32, 16)
    ov = jnp.zeros((16,), jnp.float32)
    for r, s in outs:
        ov = jnp.where(io == r, s, ov)
    o_v[...] = ov
    pltpu.sync_copy(o_v, out_hbm.at[w])


def kernel(values, positive_mask, negative_mask):
    keys, posrow, counts, hist1 = _c1(values, positive_mask, negative_mask)
    hist2, sel2 = _c2(counts, hist1, keys)
    hist3, sel3 = _c3(sel2, hist2, keys)
    rowties, sel4 = _c4(sel3, hist3, keys)
    out = _c5(sel4, rowties, posrow, keys, values)
    return out[:, :_RPW].reshape(_R)
